# Initial kernel scaffold; baseline (speedup 1.0000x reference)
import jax
import jax.numpy as jnp
from jax.experimental import pallas as pl


def kernel(boxes, scores):
    raise NotImplementedError("write your pallas kernel here")


# speedup vs baseline: 422.9243x; 422.9243x over previous
import functools

import jax
import jax.numpy as jnp
from jax.experimental import pallas as pl
from jax.experimental.pallas import tpu as pltpu

_NMS_T = 0.3
_SCORE_T = 0.1
_K = 100
_LANES = 128


def _nms_body(x_ref, s_ref, out_ref):
    x0 = x_ref[0]
    y0 = x_ref[1]
    x1 = x_ref[2]
    y1 = x_ref[3]
    sc = s_ref[...]
    shp = sc.shape
    valid = (x1 > x0) & (y1 > y0) & (sc > _SCORE_T)
    neg_inf = jnp.float32(-jnp.inf)
    s0 = jnp.where(valid, sc, neg_inf)
    area = jnp.maximum(x1 - x0, 0.0) * jnp.maximum(y1 - y0, 0.0)
    lin = (jax.lax.broadcasted_iota(jnp.int32, shp, 0) * shp[1]
           + jax.lax.broadcasted_iota(jnp.int32, shp, 1))
    lane = jax.lax.broadcasted_iota(jnp.int32, (1, _LANES), 1)

    def round_fn(k, s):
        m = jnp.max(s)
        pred = s == m
        big = jnp.int32(2**30)
        wmin = jnp.min(jnp.where(pred, lin, big))
        sel = lin == wmin
        wx0 = jnp.sum(jnp.where(sel, x0, 0.0))
        wy0 = jnp.sum(jnp.where(sel, y0, 0.0))
        wx1 = jnp.sum(jnp.where(sel, x1, 0.0))
        wy1 = jnp.sum(jnp.where(sel, y1, 0.0))
        finite = m > neg_inf
        wx0 = jnp.where(finite, wx0, 0.0)
        wy0 = jnp.where(finite, wy0, 0.0)
        wx1 = jnp.where(finite, wx1, 0.0)
        wy1 = jnp.where(finite, wy1, 0.0)
        w_area = jnp.maximum(wx1 - wx0, 0.0) * jnp.maximum(wy1 - wy0, 0.0)
        ix0 = jnp.maximum(x0, wx0)
        iy0 = jnp.maximum(y0, wy0)
        ix1 = jnp.minimum(x1, wx1)
        iy1 = jnp.minimum(y1, wy1)
        inter = jnp.maximum(ix1 - ix0, 0.0) * jnp.maximum(iy1 - iy0, 0.0)
        union = area + w_area - inter
        iou = inter / jnp.maximum(union, 1e-9)
        s = jnp.where(iou > _NMS_T, neg_inf, s)
        ms = jnp.where(finite, m, 0.0)
        row = jnp.where(lane == 0, wx0,
              jnp.where(lane == 1, wy0,
              jnp.where(lane == 2, wx1,
              jnp.where(lane == 3, wy1,
              jnp.where(lane == 4, ms, 0.0)))))
        out_ref[pl.ds(k, 1), :] = row
        return s

    jax.lax.fori_loop(0, _K, round_fn, s0)


@jax.jit
def kernel(boxes, scores):
    n = boxes.shape[0]
    rows = (n + _LANES - 1) // _LANES
    pad = rows * _LANES - n
    bt = jnp.pad(boxes.T, ((0, 0), (0, pad))).reshape(4, rows, _LANES)
    sp = jnp.pad(scores, (0, pad)).reshape(rows, _LANES)
    out = pl.pallas_call(
        _nms_body,
        out_shape=jax.ShapeDtypeStruct((_K, _LANES), jnp.float32),
        in_specs=[
            pl.BlockSpec(memory_space=pltpu.VMEM),
            pl.BlockSpec(memory_space=pltpu.VMEM),
        ],
        out_specs=pl.BlockSpec(memory_space=pltpu.VMEM),
    )(bt, sp)
    return out[:, :5]
